# baseline (speedup 1.0000x reference)
import jax
import jax.numpy as jnp
from jax.experimental import pallas as pl

_TOPK = 8
_NCOL = 64
_NROW = 32768
_BLOCK = 4096


def _router_block(xt_ref, epst_ref, outt_ref, idxt_ref):
    x = xt_ref[...]
    eps = epst_ref[...]
    noisy = x + eps * jax.nn.softplus(x)
    rows = jax.lax.broadcasted_iota(jnp.int32, noisy.shape, 0).astype(
        jnp.float32)
    work = noisy
    vals = []
    idxs = []
    for _ in range(_TOPK):
        m = jnp.max(work, axis=0, keepdims=True)
        sel = jnp.min(jnp.where(work == m, rows, float(_NCOL)), axis=0,
                      keepdims=True)
        vals.append(m)
        idxs.append(sel)
        work = jnp.where(rows == sel, -jnp.inf, work)
    v = jnp.concatenate(vals, axis=0)
    fi = jnp.concatenate(idxs, axis=0)
    p = jnp.exp(v - v[0:1])
    p = p / jnp.sum(p, axis=0, keepdims=True)
    out = jnp.zeros_like(x)
    for k in range(_TOPK):
        out = jnp.where(rows == fi[k : k + 1], p[k : k + 1], out)
    outt_ref[...] = out
    idxt_ref[...] = fi.astype(jnp.int32)


def kernel(logits):
    eps_t = jax.random.normal(
        jax.random.key(42), logits.shape, dtype=logits.dtype
    ).T
    xt = logits.T
    grid = (_NROW // _BLOCK,)
    router_t, idx_t = pl.pallas_call(
        _router_block,
        grid=grid,
        in_specs=[
            pl.BlockSpec((_NCOL, _BLOCK), lambda i: (0, i)),
            pl.BlockSpec((_NCOL, _BLOCK), lambda i: (0, i)),
        ],
        out_specs=[
            pl.BlockSpec((_NCOL, _BLOCK), lambda i: (0, i)),
            pl.BlockSpec((_TOPK, _BLOCK), lambda i: (0, i)),
        ],
        out_shape=[
            jax.ShapeDtypeStruct((_NCOL, _NROW), logits.dtype),
            jax.ShapeDtypeStruct((_TOPK, _NROW), jnp.int32),
        ],
    )(xt, eps_t)
    return router_t.T, idx_t.T
